# baseline (speedup 1.0000x reference)
import functools
import jax
import jax.numpy as jnp
from jax.experimental import pallas as pl
from jax.experimental.pallas import tpu as pltpu
from jax.experimental.pallas import tpu_sc as plsc

_B, _N, _CP = 8, 4096, 61
_NQ, _NS = 512, 32
_CIN = _CP + 3
_EPS = 1e-5
_NROWS = _B * _NQ * _NS
_RPB = _NS * _NQ

_NW = 32
_WROWS = _NROWS // _NW
_GSZ = 128
_NSTR = _WROWS // _GSZ
_GRP = 4
_CT = 128


def _fps_kernel(xs_ref, ys_ref, zs_ref, qx_ref, qy_ref, qz_ref):
    xs = xs_ref[...]
    ys = ys_ref[...]
    zs = zs_ref[...]
    lane_n = jax.lax.broadcasted_iota(jnp.int32, (_B, _N), 1)
    lane_q = jax.lax.broadcasted_iota(jnp.int32, (_B, _NQ), 1)

    def step(i, carry):
        dist_acc, fidx, qx, qy, qz = carry
        sel = lane_n == fidx
        cx = jnp.sum(jnp.where(sel, xs, 0.0), axis=1, keepdims=True)
        cy = jnp.sum(jnp.where(sel, ys, 0.0), axis=1, keepdims=True)
        cz = jnp.sum(jnp.where(sel, zs, 0.0), axis=1, keepdims=True)
        rec = lane_q == i
        qx = jnp.where(rec, cx, qx)
        qy = jnp.where(rec, cy, qy)
        qz = jnp.where(rec, cz, qz)
        dx = xs - cx
        dy = ys - cy
        dz = zs - cz
        d = dx * dx + dy * dy + dz * dz
        dist_acc = jnp.minimum(dist_acc, d)
        m = jnp.max(dist_acc, axis=1, keepdims=True)
        fidx = jnp.min(jnp.where(dist_acc == m, lane_n, _N),
                       axis=1, keepdims=True)
        return dist_acc, fidx, qx, qy, qz

    init = (jnp.full((_B, _N), 1e10, dtype=jnp.float32),
            jnp.zeros((_B, 1), dtype=jnp.int32),
            jnp.zeros((_B, _NQ), dtype=jnp.float32),
            jnp.zeros((_B, _NQ), dtype=jnp.float32),
            jnp.zeros((_B, _NQ), dtype=jnp.float32))
    _, _, qx, qy, qz = jax.lax.fori_loop(0, _NQ, step, init)
    qx_ref[...] = qx
    qy_ref[...] = qy
    qz_ref[...] = qz


def _select_kernel(qt_ref, k_ref, idx_ref):
    b = pl.program_id(0)
    qt = qt_ref[0]
    km = k_ref[0]

    dots = jax.lax.dot_general(km, qt, (((1,), (0,)), ((), ())),
                               preferred_element_type=jnp.float32)
    q2 = jnp.sum(qt * qt, axis=0, keepdims=True)
    k2 = jnp.sum(km * km, axis=1, keepdims=True)
    d2 = (q2 + k2) - 2.0 * dots

    sub_n = jax.lax.broadcasted_iota(jnp.int32, (_N, _NQ), 0)
    big = jnp.float32(3.0e38)

    rows = []
    for _ in range(_NS):
        m = jnp.min(d2, axis=0, keepdims=True)
        idx = jnp.min(jnp.where(d2 == m, sub_n, _N), axis=0, keepdims=True)
        d2 = jnp.where(sub_n == idx, big, d2)
        rows.append(idx)
    idx_ref[0] = jnp.concatenate(rows, axis=0) + b * _N


def _sc_gather_kernel(tab_ref, idx_ref, out_ref, idx_v, rows_v, sem):
    wid = jax.lax.axis_index("s") * 2 + jax.lax.axis_index("c")
    pltpu.sync_copy(idx_ref.at[wid], idx_v)
    base = wid * _WROWS
    for g in range(_NSTR // _GRP):
        copies = []
        for i in range(_GRP):
            copies.append(pltpu.async_copy(
                tab_ref.at[idx_v.at[g * _GRP + i]],
                rows_v.at[pl.ds(i * _GSZ, _GSZ)], sem))
        for c in copies:
            c.wait()
        pltpu.sync_copy(
            rows_v, out_ref.at[pl.ds(base + g * _GRP * _GSZ, _GRP * _GSZ)])


def _center_kernel(x_ref, qpad_ref, y_ref, mom_ref):
    b = pl.program_id(0)
    x = x_ref[0][:, 0:_CIN]
    qp = qpad_ref[0]
    yc = (x.reshape(_NS, _NQ, _CIN) - qp[None]).reshape(_RPB, _CIN)
    y_ref[0] = yc
    m2 = jax.lax.dot_general(yc, yc, (((0,), (0,)), ((), ())),
                             preferred_element_type=jnp.float32)
    m1 = jnp.sum(yc, axis=0, keepdims=True)

    @pl.when(b == 0)
    def _():
        mom_ref[...] = jnp.zeros_like(mom_ref)

    mom_ref[0:_CIN, :] += m2
    mom_ref[_CIN:_CIN + 1, :] += m1


def _bn_scale_shift(mom_ref, wt, bvec, gvec, bevec, cin):
    m2 = mom_ref[0:cin, :]
    m1 = mom_ref[cin:cin + 1, :]
    n = jnp.float32(_NROWS)
    a = jax.lax.dot_general(m2, wt, (((1,), (0,)), ((), ())),
                            preferred_element_type=jnp.float32)
    diag = jnp.sum(a * wt, axis=0, keepdims=True)
    wm1 = jax.lax.dot_general(m1, wt, (((1,), (0,)), ((), ())),
                              preferred_element_type=jnp.float32)
    mean = (wm1 + n * bvec) / n
    ez2 = (diag + 2.0 * bvec * wm1 + n * bvec * bvec) / n
    var = ez2 - mean * mean
    s = gvec * jax.lax.rsqrt(var + _EPS)
    t = (bvec - mean) * s + bevec
    return s, t


def _layer_kernel(x_ref, wt_ref, p_ref, mom_ref, y_ref, momout_ref, *, cin, cout):
    b = pl.program_id(0)
    wt = wt_ref[...]
    bvec = p_ref[0:1, :]
    gvec = p_ref[1:2, :]
    bevec = p_ref[2:3, :]
    s, t = _bn_scale_shift(mom_ref, wt, bvec, gvec, bevec, cin)
    z = jax.lax.dot_general(x_ref[0], wt, (((1,), (0,)), ((), ())),
                            preferred_element_type=jnp.float32)
    y = jnp.maximum(z * s + t, 0.0)
    y_ref[0] = y

    m2 = jax.lax.dot_general(y, y, (((0,), (0,)), ((), ())),
                             preferred_element_type=jnp.float32)
    m1 = jnp.sum(y, axis=0, keepdims=True)

    @pl.when(b == 0)
    def _():
        momout_ref[...] = jnp.zeros_like(momout_ref)

    momout_ref[0:cout, :] += m2
    momout_ref[cout:cout + 1, :] += m1


def _last_layer_kernel(x_ref, wt_ref, p_ref, mom_ref, o_ref, *, cin, cout):
    wt = wt_ref[...]
    bvec = p_ref[0:1, :]
    gvec = p_ref[1:2, :]
    bevec = p_ref[2:3, :]
    s, t = _bn_scale_shift(mom_ref, wt, bvec, gvec, bevec, cin)
    z = jax.lax.dot_general(x_ref[0], wt, (((1,), (0,)), ((), ())),
                            preferred_element_type=jnp.float32)
    y = jnp.maximum(z * s + t, 0.0)
    pooled = y[0:_NQ, :]
    for smp in range(1, _NS):
        pooled = jnp.maximum(pooled, y[smp * _NQ:(smp + 1) * _NQ, :])
    o_ref[0] = pooled


def _pack_params(bvec, gvec, bevec, cout):
    p = jnp.zeros((8, cout), dtype=jnp.float32)
    p = p.at[0].set(bvec).at[1].set(gvec).at[2].set(bevec)
    return p


@jax.jit
def kernel(xyz, points, W0, b0, g0, be0, W1, b1, g1, be1, W2, b2, g2, be2):
    xs = xyz[:, :, 0]
    ys = xyz[:, :, 1]
    zs = xyz[:, :, 2]
    qx, qy, qz = pl.pallas_call(
        _fps_kernel,
        out_shape=[jax.ShapeDtypeStruct((_B, _NQ), jnp.float32)] * 3,
    )(xs, ys, zs)
    new_xyz = jnp.stack([qx, qy, qz], axis=-1)
    qt = jnp.stack([qx, qy, qz], axis=1)

    gidx = pl.pallas_call(
        _select_kernel,
        grid=(_B,),
        in_specs=[
            pl.BlockSpec((1, 3, _NQ), lambda b: (b, 0, 0)),
            pl.BlockSpec((1, _N, 3), lambda b: (b, 0, 0)),
        ],
        out_specs=pl.BlockSpec((1, _NS, _NQ), lambda b: (b, 0, 0)),
        out_shape=jax.ShapeDtypeStruct((_B, _NS, _NQ), jnp.int32),
    )(qt, xyz)

    table = jnp.concatenate(
        [xyz, points, jnp.zeros((_B, _N, _CT - _CIN), jnp.float32)],
        axis=-1).reshape(_B * _N, _CT)
    idx3 = gidx.reshape(_NW, _NSTR, _GSZ)

    mesh = plsc.VectorSubcoreMesh(core_axis_name="c", subcore_axis_name="s")
    gathered = pl.kernel(
        _sc_gather_kernel,
        out_type=jax.ShapeDtypeStruct((_NROWS, _CT), jnp.float32),
        mesh=mesh,
        scratch_types=[
            pltpu.VMEM((_NSTR, _GSZ), jnp.int32),
            pltpu.VMEM((_GRP * _GSZ, _CT), jnp.float32),
            pltpu.SemaphoreType.DMA,
        ],
    )(table, idx3)

    qpad = jnp.concatenate(
        [new_xyz, jnp.zeros((_B, _NQ, _CP), jnp.float32)], axis=-1)

    x, mom0 = pl.pallas_call(
        _center_kernel,
        grid=(_B,),
        in_specs=[
            pl.BlockSpec((1, _RPB, _CT), lambda b: (b, 0, 0)),
            pl.BlockSpec((1, _NQ, _CIN), lambda b: (b, 0, 0)),
        ],
        out_specs=[
            pl.BlockSpec((1, _RPB, _CIN), lambda b: (b, 0, 0)),
            pl.BlockSpec((_CIN + 8, _CIN), lambda b: (0, 0)),
        ],
        out_shape=[
            jax.ShapeDtypeStruct((_B, _RPB, _CIN), jnp.float32),
            jax.ShapeDtypeStruct((_CIN + 8, _CIN), jnp.float32),
        ],
    )(gathered.reshape(_B, _RPB, _CT), qpad)

    def run_layer(xin, wmat, bvec, gvec, bevec, mom, cin, cout):
        return pl.pallas_call(
            functools.partial(_layer_kernel, cin=cin, cout=cout),
            grid=(_B,),
            in_specs=[
                pl.BlockSpec((1, _RPB, cin), lambda b: (b, 0, 0)),
                pl.BlockSpec((cin, cout), lambda b: (0, 0)),
                pl.BlockSpec((8, cout), lambda b: (0, 0)),
                pl.BlockSpec((cin + 8, cin), lambda b: (0, 0)),
            ],
            out_specs=[
                pl.BlockSpec((1, _RPB, cout), lambda b: (b, 0, 0)),
                pl.BlockSpec((cout + 8, cout), lambda b: (0, 0)),
            ],
            out_shape=[
                jax.ShapeDtypeStruct((_B, _RPB, cout), jnp.float32),
                jax.ShapeDtypeStruct((cout + 8, cout), jnp.float32),
            ],
        )(xin, wmat.T, _pack_params(bvec, gvec, bevec, cout), mom)

    y1, mom1 = run_layer(x, W0, b0, g0, be0, mom0, _CIN, 128)
    y2, mom2 = run_layer(y1, W1, b1, g1, be1, mom1, 128, 128)

    out = pl.pallas_call(
        functools.partial(_last_layer_kernel, cin=128, cout=256),
        grid=(_B,),
        in_specs=[
            pl.BlockSpec((1, _RPB, 128), lambda b: (b, 0, 0)),
            pl.BlockSpec((128, 256), lambda b: (0, 0)),
            pl.BlockSpec((8, 256), lambda b: (0, 0)),
            pl.BlockSpec((128 + 8, 128), lambda b: (0, 0)),
        ],
        out_specs=pl.BlockSpec((1, _NQ, 256), lambda b: (b, 0, 0)),
        out_shape=jax.ShapeDtypeStruct((_B, _NQ, 256), jnp.float32),
    )(y2, W2.T, _pack_params(b2, g2, be2, 256), mom2)

    return new_xyz, jnp.transpose(out, (0, 2, 1))


# speedup vs baseline: 1.0220x; 1.0220x over previous
import functools
import jax
import jax.numpy as jnp
from jax.experimental import pallas as pl
from jax.experimental.pallas import tpu as pltpu
from jax.experimental.pallas import tpu_sc as plsc

_B, _N, _CP = 8, 4096, 61
_NQ, _NS = 512, 32
_CIN = _CP + 3
_EPS = 1e-5
_NROWS = _B * _NQ * _NS
_RPB = _NS * _NQ

_NW = 32
_WROWS = _NROWS // _NW
_GSZ = 128
_NSTR = _WROWS // _GSZ
_GRP = 4
_CT = 128


def _fps_kernel(xs_ref, ys_ref, zs_ref, qx_ref, qy_ref, qz_ref):
    xs = xs_ref[...]
    ys = ys_ref[...]
    zs = zs_ref[...]
    lane_n = jax.lax.broadcasted_iota(jnp.int32, (_B, _N), 1)
    lane_q = jax.lax.broadcasted_iota(jnp.int32, (_B, _NQ), 1)

    def step(i, carry):
        dist_acc, fidx, qx, qy, qz = carry
        sel = lane_n == fidx
        cx = jnp.sum(jnp.where(sel, xs, 0.0), axis=1, keepdims=True)
        cy = jnp.sum(jnp.where(sel, ys, 0.0), axis=1, keepdims=True)
        cz = jnp.sum(jnp.where(sel, zs, 0.0), axis=1, keepdims=True)
        rec = lane_q == i
        qx = jnp.where(rec, cx, qx)
        qy = jnp.where(rec, cy, qy)
        qz = jnp.where(rec, cz, qz)
        dx = xs - cx
        dy = ys - cy
        dz = zs - cz
        d = dx * dx + dy * dy + dz * dz
        dist_acc = jnp.minimum(dist_acc, d)
        m = jnp.max(dist_acc, axis=1, keepdims=True)
        fidx = jnp.min(jnp.where(dist_acc == m, lane_n, _N),
                       axis=1, keepdims=True)
        return dist_acc, fidx, qx, qy, qz

    init = (jnp.full((_B, _N), 1e10, dtype=jnp.float32),
            jnp.zeros((_B, 1), dtype=jnp.int32),
            jnp.zeros((_B, _NQ), dtype=jnp.float32),
            jnp.zeros((_B, _NQ), dtype=jnp.float32),
            jnp.zeros((_B, _NQ), dtype=jnp.float32))
    _, _, qx, qy, qz = jax.lax.fori_loop(0, _NQ, step, init)
    qx_ref[...] = qx
    qy_ref[...] = qy
    qz_ref[...] = qz


def _select_kernel(qt_ref, k_ref, idx_ref):
    b = pl.program_id(0)
    qt = qt_ref[0]
    km = k_ref[0]

    dots = jax.lax.dot_general(km, qt, (((1,), (0,)), ((), ())),
                               preferred_element_type=jnp.float32)
    q2 = jnp.sum(qt * qt, axis=0, keepdims=True)
    k2 = jnp.sum(km * km, axis=1, keepdims=True)
    d2 = (q2 + k2) - 2.0 * dots

    sub_n = jax.lax.broadcasted_iota(jnp.int32, (_N, _NQ), 0)
    big = jnp.float32(3.0e38)

    rows = []
    for _ in range(_NS):
        m = jnp.min(d2, axis=0, keepdims=True)
        msk = d2 == m
        idx = jnp.min(jnp.where(msk, sub_n, _N), axis=0, keepdims=True)
        d2 = jnp.where(msk, big, d2)
        rows.append(idx)
    idx_ref[0] = jnp.concatenate(rows, axis=0) + b * _N


def _sc_gather_kernel(tab_ref, idx_ref, out_ref, idx_v, rows_v, sem):
    wid = jax.lax.axis_index("s") * 2 + jax.lax.axis_index("c")
    pltpu.sync_copy(idx_ref.at[wid], idx_v)
    base = wid * _WROWS
    for g in range(_NSTR // _GRP):
        copies = []
        for i in range(_GRP):
            copies.append(pltpu.async_copy(
                tab_ref.at[idx_v.at[g * _GRP + i]],
                rows_v.at[pl.ds(i * _GSZ, _GSZ)], sem))
        for c in copies:
            c.wait()
        pltpu.sync_copy(
            rows_v, out_ref.at[pl.ds(base + g * _GRP * _GSZ, _GRP * _GSZ)])


def _center_kernel(x_ref, qpad_ref, y_ref, mom_ref):
    b = pl.program_id(0)
    x = x_ref[0][:, 0:_CIN]
    qp = qpad_ref[0]
    yc = (x.reshape(_NS, _NQ, _CIN) - qp[None]).reshape(_RPB, _CIN)
    y_ref[0] = yc
    m2 = jax.lax.dot_general(yc, yc, (((0,), (0,)), ((), ())),
                             preferred_element_type=jnp.float32)
    m1 = jnp.sum(yc, axis=0, keepdims=True)

    @pl.when(b == 0)
    def _():
        mom_ref[...] = jnp.zeros_like(mom_ref)

    mom_ref[0:_CIN, :] += m2
    mom_ref[_CIN:_CIN + 1, :] += m1


def _bn_scale_shift(mom_ref, wt, bvec, gvec, bevec, cin):
    m2 = mom_ref[0:cin, :]
    m1 = mom_ref[cin:cin + 1, :]
    n = jnp.float32(_NROWS)
    a = jax.lax.dot_general(m2, wt, (((1,), (0,)), ((), ())),
                            preferred_element_type=jnp.float32)
    diag = jnp.sum(a * wt, axis=0, keepdims=True)
    wm1 = jax.lax.dot_general(m1, wt, (((1,), (0,)), ((), ())),
                              preferred_element_type=jnp.float32)
    mean = (wm1 + n * bvec) / n
    ez2 = (diag + 2.0 * bvec * wm1 + n * bvec * bvec) / n
    var = ez2 - mean * mean
    s = gvec * jax.lax.rsqrt(var + _EPS)
    t = (bvec - mean) * s + bevec
    return s, t


def _layer_kernel(x_ref, wt_ref, p_ref, mom_ref, y_ref, momout_ref, *, cin, cout):
    b = pl.program_id(0)
    wt = wt_ref[...]
    bvec = p_ref[0:1, :]
    gvec = p_ref[1:2, :]
    bevec = p_ref[2:3, :]
    s, t = _bn_scale_shift(mom_ref, wt, bvec, gvec, bevec, cin)
    z = jax.lax.dot_general(x_ref[0], wt, (((1,), (0,)), ((), ())),
                            preferred_element_type=jnp.float32)
    y = jnp.maximum(z * s + t, 0.0)
    y_ref[0] = y

    m2 = jax.lax.dot_general(y, y, (((0,), (0,)), ((), ())),
                             preferred_element_type=jnp.float32)
    m1 = jnp.sum(y, axis=0, keepdims=True)

    @pl.when(b == 0)
    def _():
        momout_ref[...] = jnp.zeros_like(momout_ref)

    momout_ref[0:cout, :] += m2
    momout_ref[cout:cout + 1, :] += m1


def _last_layer_kernel(x_ref, wt_ref, p_ref, mom_ref, o_ref, *, cin, cout):
    wt = wt_ref[...]
    bvec = p_ref[0:1, :]
    gvec = p_ref[1:2, :]
    bevec = p_ref[2:3, :]
    s, t = _bn_scale_shift(mom_ref, wt, bvec, gvec, bevec, cin)
    z = jax.lax.dot_general(x_ref[0], wt, (((1,), (0,)), ((), ())),
                            preferred_element_type=jnp.float32)
    y = jnp.maximum(z * s + t, 0.0)
    pooled = y[0:_NQ, :]
    for smp in range(1, _NS):
        pooled = jnp.maximum(pooled, y[smp * _NQ:(smp + 1) * _NQ, :])
    o_ref[0] = pooled


def _pack_params(bvec, gvec, bevec, cout):
    p = jnp.zeros((8, cout), dtype=jnp.float32)
    p = p.at[0].set(bvec).at[1].set(gvec).at[2].set(bevec)
    return p


@jax.jit
def kernel(xyz, points, W0, b0, g0, be0, W1, b1, g1, be1, W2, b2, g2, be2):
    xs = xyz[:, :, 0]
    ys = xyz[:, :, 1]
    zs = xyz[:, :, 2]
    qx, qy, qz = pl.pallas_call(
        _fps_kernel,
        out_shape=[jax.ShapeDtypeStruct((_B, _NQ), jnp.float32)] * 3,
    )(xs, ys, zs)
    new_xyz = jnp.stack([qx, qy, qz], axis=-1)
    qt = jnp.stack([qx, qy, qz], axis=1)

    gidx = pl.pallas_call(
        _select_kernel,
        grid=(_B,),
        in_specs=[
            pl.BlockSpec((1, 3, _NQ), lambda b: (b, 0, 0)),
            pl.BlockSpec((1, _N, 3), lambda b: (b, 0, 0)),
        ],
        out_specs=pl.BlockSpec((1, _NS, _NQ), lambda b: (b, 0, 0)),
        out_shape=jax.ShapeDtypeStruct((_B, _NS, _NQ), jnp.int32),
    )(qt, xyz)

    table = jnp.concatenate(
        [xyz, points, jnp.zeros((_B, _N, _CT - _CIN), jnp.float32)],
        axis=-1).reshape(_B * _N, _CT)
    idx3 = gidx.reshape(_NW, _NSTR, _GSZ)

    mesh = plsc.VectorSubcoreMesh(core_axis_name="c", subcore_axis_name="s")
    gathered = pl.kernel(
        _sc_gather_kernel,
        out_type=jax.ShapeDtypeStruct((_NROWS, _CT), jnp.float32),
        mesh=mesh,
        scratch_types=[
            pltpu.VMEM((_NSTR, _GSZ), jnp.int32),
            pltpu.VMEM((_GRP * _GSZ, _CT), jnp.float32),
            pltpu.SemaphoreType.DMA,
        ],
    )(table, idx3)

    qpad = jnp.concatenate(
        [new_xyz, jnp.zeros((_B, _NQ, _CP), jnp.float32)], axis=-1)

    x, mom0 = pl.pallas_call(
        _center_kernel,
        grid=(_B,),
        in_specs=[
            pl.BlockSpec((1, _RPB, _CT), lambda b: (b, 0, 0)),
            pl.BlockSpec((1, _NQ, _CIN), lambda b: (b, 0, 0)),
        ],
        out_specs=[
            pl.BlockSpec((1, _RPB, _CIN), lambda b: (b, 0, 0)),
            pl.BlockSpec((_CIN + 8, _CIN), lambda b: (0, 0)),
        ],
        out_shape=[
            jax.ShapeDtypeStruct((_B, _RPB, _CIN), jnp.float32),
            jax.ShapeDtypeStruct((_CIN + 8, _CIN), jnp.float32),
        ],
    )(gathered.reshape(_B, _RPB, _CT), qpad)

    def run_layer(xin, wmat, bvec, gvec, bevec, mom, cin, cout):
        return pl.pallas_call(
            functools.partial(_layer_kernel, cin=cin, cout=cout),
            grid=(_B,),
            in_specs=[
                pl.BlockSpec((1, _RPB, cin), lambda b: (b, 0, 0)),
                pl.BlockSpec((cin, cout), lambda b: (0, 0)),
                pl.BlockSpec((8, cout), lambda b: (0, 0)),
                pl.BlockSpec((cin + 8, cin), lambda b: (0, 0)),
            ],
            out_specs=[
                pl.BlockSpec((1, _RPB, cout), lambda b: (b, 0, 0)),
                pl.BlockSpec((cout + 8, cout), lambda b: (0, 0)),
            ],
            out_shape=[
                jax.ShapeDtypeStruct((_B, _RPB, cout), jnp.float32),
                jax.ShapeDtypeStruct((cout + 8, cout), jnp.float32),
            ],
        )(xin, wmat.T, _pack_params(bvec, gvec, bevec, cout), mom)

    y1, mom1 = run_layer(x, W0, b0, g0, be0, mom0, _CIN, 128)
    y2, mom2 = run_layer(y1, W1, b1, g1, be1, mom1, 128, 128)

    out = pl.pallas_call(
        functools.partial(_last_layer_kernel, cin=128, cout=256),
        grid=(_B,),
        in_specs=[
            pl.BlockSpec((1, _RPB, 128), lambda b: (b, 0, 0)),
            pl.BlockSpec((128, 256), lambda b: (0, 0)),
            pl.BlockSpec((8, 256), lambda b: (0, 0)),
            pl.BlockSpec((128 + 8, 128), lambda b: (0, 0)),
        ],
        out_specs=pl.BlockSpec((1, _NQ, 256), lambda b: (b, 0, 0)),
        out_shape=jax.ShapeDtypeStruct((_B, _NQ, 256), jnp.float32),
    )(y2, W2.T, _pack_params(b2, g2, be2, 256), mom2)

    return new_xyz, jnp.transpose(out, (0, 2, 1))
